# Initial kernel scaffold; baseline (speedup 1.0000x reference)
import jax
import jax.numpy as jnp
from jax.experimental import pallas as pl


def kernel(unary, image):
    raise NotImplementedError("write your pallas kernel here")


# speedup vs baseline: 1.8345x; 1.8345x over previous
import functools

import jax
import jax.numpy as jnp
from jax import lax
from jax.experimental import pallas as pl
from jax.experimental.pallas import tpu as pltpu
from jax.experimental.pallas import tpu_sc as plsc

H = 512
W = 512
C = 21
CP = 24
N = H * W
M = 1 << 18
THETA_ALPHA = 80.0
THETA_BETA = 13.0
THETA_GAMMA = 3.0
BI_COMPAT = 10.0
SP_COMPAT = 3.0
NUM_ITER = 5

_BIP = jnp.array([2654435761, 805459861, 3674653429, 2097192037, 1434869437],
                 dtype=jnp.uint32)
_SPP = jnp.array([2654435761, 805459861], dtype=jnp.uint32)

NS = 16
CHUNK = 1 << 16
NCHUNK = M // CHUNK
GR = 256
ROWS = CHUNK + GR
ZROWS = ROWS // NS
ZB = 257
TROWS = CHUNK // NS
PXT = N // NS
BLK = 512
NBLK = PXT // BLK
IB = BLK // 128
IROWS = N // 128


def _sc_body(q_hbm, bidx_hbm, sidx_hbm, big_hbm, spg_hbm, btab_hbm, stab_hbm,
             chunk_sh, qblk, gidx, lidx, zbuf, sem):
    c = lax.axis_index("c")
    s = lax.axis_index("s")

    zv = jnp.zeros((16,), jnp.float32)

    def _zz(r, carry):
        zbuf[r, pl.ds(0, 16)] = zv
        zbuf[r, pl.ds(8, 16)] = zv
        return carry

    lax.fori_loop(0, ZB, _zz, 0)

    def do_table(idx_hbm, tab_hbm, out_hbm):
        for ci in range(NCHUNK):
            base = ci * CHUNK
            for zi in range(16):
                pltpu.sync_copy(zbuf, chunk_sh.at[pl.ds(s * ZROWS + zi * ZB, ZB)])
            plsc.subcore_barrier()

            def _sblk(b, carry):
                row0 = s * (PXT // 128) + b * IB
                px0 = s * PXT + b * BLK
                pltpu.sync_copy(idx_hbm.at[pl.ds(row0, IB)], gidx)
                pltpu.sync_copy(q_hbm.at[pl.ds(px0, BLK)], qblk)

                def _lk(k, kc):
                    r = k // 8
                    o = (k % 8) * 16
                    v = gidx[r, pl.ds(o, 16)]
                    m = (v >= base) & (v < base + CHUNK)
                    g = CHUNK + ((k * 16 + lax.iota(jnp.int32, 16)) & (GR - 1))
                    lidx[r, pl.ds(o, 16)] = jnp.where(m, v - base, g)
                    return kc

                lax.fori_loop(0, 8 * IB, _lk, 0)
                for j in range(IB):
                    pltpu.sync_copy(qblk.at[pl.ds(j * 128, 128)],
                                    chunk_sh.at[lidx.at[j]], add=True)
                return carry

            lax.fori_loop(0, NBLK, _sblk, 0)
            plsc.subcore_barrier()
            pltpu.sync_copy(chunk_sh.at[pl.ds(s * TROWS, TROWS)],
                            tab_hbm.at[pl.ds(base + s * TROWS, TROWS)])
            plsc.subcore_barrier()

        def _gblk(b, carry):
            row0 = s * (PXT // 128) + b * IB
            px0 = s * PXT + b * BLK
            pltpu.sync_copy(idx_hbm.at[pl.ds(row0, IB)], gidx)
            handles = []
            for j in range(IB):
                handles.append(pltpu.async_copy(tab_hbm.at[gidx.at[j]],
                                                qblk.at[pl.ds(j * 128, 128)],
                                                sem))
            for h in handles:
                h.wait()
            pltpu.sync_copy(qblk, out_hbm.at[pl.ds(px0, BLK)])
            return carry

        lax.fori_loop(0, NBLK, _gblk, 0)

    @pl.when(c == 0)
    def _():
        do_table(bidx_hbm, btab_hbm, big_hbm)

    @pl.when(c == 1)
    def _():
        do_table(sidx_hbm, stab_hbm, spg_hbm)


@functools.lru_cache(maxsize=1)
def _sc_splat_slice():
  return pl.kernel(
    _sc_body,
    mesh=plsc.VectorSubcoreMesh(core_axis_name="c", subcore_axis_name="s"),
    out_type=[
        jax.ShapeDtypeStruct((N, CP), jnp.float32),
        jax.ShapeDtypeStruct((N, CP), jnp.float32),
        jax.ShapeDtypeStruct((M, CP), jnp.float32),
        jax.ShapeDtypeStruct((M, CP), jnp.float32),
    ],
    scratch_types=[
        pltpu.VMEM_SHARED((ROWS, CP), jnp.float32),
        pltpu.VMEM((BLK, CP), jnp.float32),
        pltpu.VMEM((IB, 128), jnp.int32),
        pltpu.VMEM((IB, 128), jnp.int32),
        pltpu.VMEM((ZB, CP), jnp.float32),
        pltpu.SemaphoreType.DMA,
    ],
    compiler_params=pltpu.CompilerParams(use_tc_tiling_on_sc=False),
  )


def _tc_iter_body(q_ref, u_ref, bg_ref, sg_ref, o_ref):
    q = q_ref[...]
    pw = BI_COMPAT * (bg_ref[...] - q) + SP_COMPAT * (sg_ref[...] - q)
    pw = jnp.sum(pw, axis=1, keepdims=True) - pw
    lg = -u_ref[...] - pw
    lg = lg - jnp.max(lg, axis=1, keepdims=True)
    e = jnp.exp(lg)
    o_ref[...] = e / jnp.sum(e, axis=1, keepdims=True)


def _tc_iter(q, upad, bg, sg):
    blk = 8192
    spec = pl.BlockSpec((blk, CP), lambda i: (i, 0))
    return pl.pallas_call(
        _tc_iter_body,
        grid=(N // blk,),
        in_specs=[spec, spec, spec, spec],
        out_specs=spec,
        out_shape=jax.ShapeDtypeStruct((N, CP), jnp.float32),
    )(q, upad, bg, sg)


def _bucket_indices(image):
    ys = jnp.arange(H, dtype=jnp.float32)
    xs = jnp.arange(W, dtype=jnp.float32)
    yy, xx = jnp.meshgrid(ys, xs, indexing="ij")
    yy = yy.reshape(-1)
    xx = xx.reshape(-1)
    img = image.reshape(-1, 3)
    bilateral = jnp.stack([xx / THETA_ALPHA, yy / THETA_ALPHA], axis=-1)
    bilateral = jnp.concatenate([bilateral, img * (255.0 / THETA_BETA)], axis=-1)
    spatial = jnp.stack([xx / THETA_GAMMA, yy / THETA_GAMMA], axis=-1)

    def lat(feats, primes):
        coords = jnp.round(feats).astype(jnp.int32).astype(jnp.uint32)
        h = (coords * primes[None, :]).sum(axis=-1)
        return (h % jnp.uint32(M)).astype(jnp.int32)

    return lat(bilateral, _BIP), lat(spatial, _SPP)


def kernel(unary, image):
    u = unary.reshape(N, C)
    upad = jnp.pad(u, ((0, 0), (0, CP - C)), constant_values=1e30)
    bi_idx, sp_idx = _bucket_indices(image)
    bi2 = bi_idx.reshape(IROWS, 128)
    sp2 = sp_idx.reshape(IROWS, 128)
    z = jnp.zeros((N, CP), jnp.float32)
    q = _tc_iter(z, upad, z, z)
    for _ in range(NUM_ITER):
        bg, sg, _, _ = _sc_splat_slice()(q, bi2, sp2)
        q = _tc_iter(q, upad, bg, sg)
    return q[:, :C].reshape(H, W, C)


# speedup vs baseline: 1.9911x; 1.0854x over previous
import functools

import jax
import jax.numpy as jnp
from jax import lax
from jax.experimental import pallas as pl
from jax.experimental.pallas import tpu as pltpu
from jax.experimental.pallas import tpu_sc as plsc

H = 512
W = 512
C = 21
CP = 24
N = H * W
M = 1 << 18
THETA_ALPHA = 80.0
THETA_BETA = 13.0
THETA_GAMMA = 3.0
BI_COMPAT = 10.0
SP_COMPAT = 3.0
NUM_ITER = 5

_BIP = jnp.array([2654435761, 805459861, 3674653429, 2097192037, 1434869437],
                 dtype=jnp.uint32)
_SPP = jnp.array([2654435761, 805459861], dtype=jnp.uint32)

NS = 16
CHUNK = 1 << 16
NCHUNK = M // CHUNK
GR = 256
ROWS = CHUNK + GR
ZROWS = ROWS // NS
ZB = 257
TROWS = CHUNK // NS
PXT = N // NS
BLK = 512
NBLK = PXT // BLK
IB = BLK // 128
IROWS = N // 128


def _sc_body(q_hbm, bidx_hbm, sidx_hbm, big_hbm, spg_hbm, btab_hbm, stab_hbm,
             chunk_sh, qblk, gidx, lidx, zbuf, sem):
    c = lax.axis_index("c")
    s = lax.axis_index("s")

    zv = jnp.zeros((16,), jnp.float32)

    def _zz(r, carry):
        zbuf[r, pl.ds(0, 16)] = zv
        zbuf[r, pl.ds(8, 16)] = zv
        return carry

    lax.fori_loop(0, ZB, _zz, 0)

    def do_table(idx_hbm, tab_hbm, out_hbm):
        for ci in range(NCHUNK):
            base = ci * CHUNK
            for zi in range(16):
                pltpu.sync_copy(zbuf, chunk_sh.at[pl.ds(s * ZROWS + zi * ZB, ZB)])
            plsc.subcore_barrier()

            def _sblk(b, carry):
                row0 = s * (PXT // 128) + b * IB
                px0 = s * PXT + b * BLK
                h1 = pltpu.async_copy(idx_hbm.at[pl.ds(row0, IB)], gidx, sem)
                h2 = pltpu.async_copy(q_hbm.at[pl.ds(px0, BLK)], qblk, sem)
                h1.wait()
                for j in range(IB):
                    def _lk(k, kc, j=j):
                        v = gidx[j, pl.ds(k * 16, 16)]
                        m = (v >= base) & (v < base + CHUNK)
                        g = CHUNK + (((j * 8 + k) * 16
                                      + lax.iota(jnp.int32, 16)) & (GR - 1))
                        lidx[j, pl.ds(k * 16, 16)] = jnp.where(m, v - base, g)
                        return kc

                    lax.fori_loop(0, 8, _lk, 0)
                h2.wait()
                for j in range(IB):
                    pltpu.sync_copy(qblk.at[pl.ds(j * 128, 128)],
                                    chunk_sh.at[lidx.at[j]], add=True)
                return carry

            lax.fori_loop(0, NBLK, _sblk, 0)
            plsc.subcore_barrier()
            pltpu.sync_copy(chunk_sh.at[pl.ds(s * TROWS, TROWS)],
                            tab_hbm.at[pl.ds(base + s * TROWS, TROWS)])
            plsc.subcore_barrier()

        def _gblk(b, carry):
            row0 = s * (PXT // 128) + b * IB
            px0 = s * PXT + b * BLK
            pltpu.sync_copy(idx_hbm.at[pl.ds(row0, IB)], gidx)
            handles = []
            for j in range(IB):
                handles.append(pltpu.async_copy(tab_hbm.at[gidx.at[j]],
                                                qblk.at[pl.ds(j * 128, 128)],
                                                sem))
            for h in handles:
                h.wait()
            pltpu.sync_copy(qblk, out_hbm.at[pl.ds(px0, BLK)])
            return carry

        lax.fori_loop(0, NBLK, _gblk, 0)

    @pl.when(c == 0)
    def _():
        do_table(bidx_hbm, btab_hbm, big_hbm)

    @pl.when(c == 1)
    def _():
        do_table(sidx_hbm, stab_hbm, spg_hbm)


@functools.lru_cache(maxsize=1)
def _sc_splat_slice():
  return pl.kernel(
    _sc_body,
    mesh=plsc.VectorSubcoreMesh(core_axis_name="c", subcore_axis_name="s"),
    out_type=[
        jax.ShapeDtypeStruct((N, CP), jnp.float32),
        jax.ShapeDtypeStruct((N, CP), jnp.float32),
        jax.ShapeDtypeStruct((M, CP), jnp.float32),
        jax.ShapeDtypeStruct((M, CP), jnp.float32),
    ],
    scratch_types=[
        pltpu.VMEM_SHARED((ROWS, CP), jnp.float32),
        pltpu.VMEM((BLK, CP), jnp.float32),
        pltpu.VMEM((IB, 128), jnp.int32),
        pltpu.VMEM((IB, 128), jnp.int32),
        pltpu.VMEM((ZB, CP), jnp.float32),
        pltpu.SemaphoreType.DMA,
    ],
    compiler_params=pltpu.CompilerParams(use_tc_tiling_on_sc=False),
  )


def _tc_iter_body(q_ref, u_ref, bg_ref, sg_ref, o_ref):
    q = q_ref[...]
    pw = BI_COMPAT * (bg_ref[...] - q) + SP_COMPAT * (sg_ref[...] - q)
    pw = jnp.sum(pw, axis=1, keepdims=True) - pw
    lg = -u_ref[...] - pw
    lg = lg - jnp.max(lg, axis=1, keepdims=True)
    e = jnp.exp(lg)
    o_ref[...] = e / jnp.sum(e, axis=1, keepdims=True)


def _tc_iter(q, upad, bg, sg):
    blk = 8192
    spec = pl.BlockSpec((blk, CP), lambda i: (i, 0))
    return pl.pallas_call(
        _tc_iter_body,
        grid=(N // blk,),
        in_specs=[spec, spec, spec, spec],
        out_specs=spec,
        out_shape=jax.ShapeDtypeStruct((N, CP), jnp.float32),
    )(q, upad, bg, sg)


def _bucket_indices(image):
    ys = jnp.arange(H, dtype=jnp.float32)
    xs = jnp.arange(W, dtype=jnp.float32)
    yy, xx = jnp.meshgrid(ys, xs, indexing="ij")
    yy = yy.reshape(-1)
    xx = xx.reshape(-1)
    img = image.reshape(-1, 3)
    bilateral = jnp.stack([xx / THETA_ALPHA, yy / THETA_ALPHA], axis=-1)
    bilateral = jnp.concatenate([bilateral, img * (255.0 / THETA_BETA)], axis=-1)
    spatial = jnp.stack([xx / THETA_GAMMA, yy / THETA_GAMMA], axis=-1)

    def lat(feats, primes):
        coords = jnp.round(feats).astype(jnp.int32).astype(jnp.uint32)
        h = (coords * primes[None, :]).sum(axis=-1)
        return (h % jnp.uint32(M)).astype(jnp.int32)

    return lat(bilateral, _BIP), lat(spatial, _SPP)


def kernel(unary, image):
    u = unary.reshape(N, C)
    upad = jnp.pad(u, ((0, 0), (0, CP - C)), constant_values=1e30)
    bi_idx, sp_idx = _bucket_indices(image)
    bi2 = bi_idx.reshape(IROWS, 128)
    sp2 = sp_idx.reshape(IROWS, 128)
    z = jnp.zeros((N, CP), jnp.float32)
    q = _tc_iter(z, upad, z, z)
    for _ in range(NUM_ITER):
        bg, sg, _, _ = _sc_splat_slice()(q, bi2, sp2)
        q = _tc_iter(q, upad, bg, sg)
    return q[:, :C].reshape(H, W, C)


# speedup vs baseline: 2.0479x; 1.0285x over previous
import functools

import jax
import jax.numpy as jnp
from jax import lax
from jax.experimental import pallas as pl
from jax.experimental.pallas import tpu as pltpu
from jax.experimental.pallas import tpu_sc as plsc

H = 512
W = 512
C = 21
CP = 24
N = H * W
M = 1 << 18
THETA_ALPHA = 80.0
THETA_BETA = 13.0
THETA_GAMMA = 3.0
BI_COMPAT = 10.0
SP_COMPAT = 3.0
NUM_ITER = 5

_BIP = jnp.array([2654435761, 805459861, 3674653429, 2097192037, 1434869437],
                 dtype=jnp.uint32)
_SPP = jnp.array([2654435761, 805459861], dtype=jnp.uint32)

NS = 16
CHUNK = 1 << 16
NCHUNK = M // CHUNK
GR = 256
ROWS = CHUNK + GR
ZROWS = ROWS // NS
ZB = 257
TROWS = CHUNK // NS
PXT = N // NS
BLK = 512
NBLK = PXT // BLK
IB = BLK // 128
IROWS = N // 128


def _sc_body(q_hbm, bidx_hbm, sidx_hbm, big_hbm, spg_hbm, btab_hbm, stab_hbm,
             chunk_sh, qblk, gidx, lidx, zbuf, sem):
    c = lax.axis_index("c")
    s = lax.axis_index("s")

    zv = jnp.zeros((16,), jnp.float32)

    def _zz(r, carry):
        zbuf[r, pl.ds(0, 16)] = zv
        zbuf[r, pl.ds(8, 16)] = zv
        return carry

    lax.fori_loop(0, ZB, _zz, 0)

    def do_table(idx_hbm, tab_hbm, out_hbm):
        for ci in range(NCHUNK):
            base = ci * CHUNK
            zh = [pltpu.async_copy(
                      zbuf, chunk_sh.at[pl.ds(s * ZROWS + zi * ZB, ZB)], sem)
                  for zi in range(16)]
            for h in zh:
                h.wait()
            plsc.subcore_barrier()

            def _sblk(b, carry):
                row0 = s * (PXT // 128) + b * IB
                px0 = s * PXT + b * BLK
                h1 = pltpu.async_copy(idx_hbm.at[pl.ds(row0, IB)], gidx, sem)
                h2 = pltpu.async_copy(q_hbm.at[pl.ds(px0, BLK)], qblk, sem)
                h1.wait()
                for j in range(IB):
                    def _lk(k, kc, j=j):
                        v = gidx[j, pl.ds(k * 16, 16)]
                        m = (v >= base) & (v < base + CHUNK)
                        g = CHUNK + (((j * 8 + k) * 16
                                      + lax.iota(jnp.int32, 16)) & (GR - 1))
                        lidx[j, pl.ds(k * 16, 16)] = jnp.where(m, v - base, g)
                        return kc

                    lax.fori_loop(0, 8, _lk, 0)
                h2.wait()
                sh = [pltpu.async_copy(qblk.at[pl.ds(j * 128, 128)],
                                       chunk_sh.at[lidx.at[j]], sem, add=True)
                      for j in range(IB)]
                for h in sh:
                    h.wait()
                return carry

            lax.fori_loop(0, NBLK, _sblk, 0)
            plsc.subcore_barrier()
            pltpu.sync_copy(chunk_sh.at[pl.ds(s * TROWS, TROWS)],
                            tab_hbm.at[pl.ds(base + s * TROWS, TROWS)])
            plsc.subcore_barrier()

        def _gblk(b, carry):
            row0 = s * (PXT // 128) + b * IB
            px0 = s * PXT + b * BLK
            pltpu.sync_copy(idx_hbm.at[pl.ds(row0, IB)], gidx)
            handles = []
            for j in range(IB):
                handles.append(pltpu.async_copy(tab_hbm.at[gidx.at[j]],
                                                qblk.at[pl.ds(j * 128, 128)],
                                                sem))
            for h in handles:
                h.wait()
            pltpu.sync_copy(qblk, out_hbm.at[pl.ds(px0, BLK)])
            return carry

        lax.fori_loop(0, NBLK, _gblk, 0)

    @pl.when(c == 0)
    def _():
        do_table(bidx_hbm, btab_hbm, big_hbm)

    @pl.when(c == 1)
    def _():
        do_table(sidx_hbm, stab_hbm, spg_hbm)


@functools.lru_cache(maxsize=1)
def _sc_splat_slice():
  return pl.kernel(
    _sc_body,
    mesh=plsc.VectorSubcoreMesh(core_axis_name="c", subcore_axis_name="s"),
    out_type=[
        jax.ShapeDtypeStruct((N, CP), jnp.float32),
        jax.ShapeDtypeStruct((N, CP), jnp.float32),
        jax.ShapeDtypeStruct((M, CP), jnp.float32),
        jax.ShapeDtypeStruct((M, CP), jnp.float32),
    ],
    scratch_types=[
        pltpu.VMEM_SHARED((ROWS, CP), jnp.float32),
        pltpu.VMEM((BLK, CP), jnp.float32),
        pltpu.VMEM((IB, 128), jnp.int32),
        pltpu.VMEM((IB, 128), jnp.int32),
        pltpu.VMEM((ZB, CP), jnp.float32),
        pltpu.SemaphoreType.DMA,
    ],
    compiler_params=pltpu.CompilerParams(use_tc_tiling_on_sc=False),
  )


def _tc_iter_body(q_ref, u_ref, bg_ref, sg_ref, o_ref):
    q = q_ref[...]
    pw = BI_COMPAT * (bg_ref[...] - q) + SP_COMPAT * (sg_ref[...] - q)
    pw = jnp.sum(pw, axis=1, keepdims=True) - pw
    lg = -u_ref[...] - pw
    lg = lg - jnp.max(lg, axis=1, keepdims=True)
    e = jnp.exp(lg)
    o_ref[...] = e / jnp.sum(e, axis=1, keepdims=True)


def _tc_iter(q, upad, bg, sg):
    blk = 8192
    spec = pl.BlockSpec((blk, CP), lambda i: (i, 0))
    return pl.pallas_call(
        _tc_iter_body,
        grid=(N // blk,),
        in_specs=[spec, spec, spec, spec],
        out_specs=spec,
        out_shape=jax.ShapeDtypeStruct((N, CP), jnp.float32),
    )(q, upad, bg, sg)


def _bucket_indices(image):
    ys = jnp.arange(H, dtype=jnp.float32)
    xs = jnp.arange(W, dtype=jnp.float32)
    yy, xx = jnp.meshgrid(ys, xs, indexing="ij")
    yy = yy.reshape(-1)
    xx = xx.reshape(-1)
    img = image.reshape(-1, 3)
    bilateral = jnp.stack([xx / THETA_ALPHA, yy / THETA_ALPHA], axis=-1)
    bilateral = jnp.concatenate([bilateral, img * (255.0 / THETA_BETA)], axis=-1)
    spatial = jnp.stack([xx / THETA_GAMMA, yy / THETA_GAMMA], axis=-1)

    def lat(feats, primes):
        coords = jnp.round(feats).astype(jnp.int32).astype(jnp.uint32)
        h = (coords * primes[None, :]).sum(axis=-1)
        return (h % jnp.uint32(M)).astype(jnp.int32)

    return lat(bilateral, _BIP), lat(spatial, _SPP)


def kernel(unary, image):
    u = unary.reshape(N, C)
    upad = jnp.pad(u, ((0, 0), (0, CP - C)), constant_values=1e30)
    bi_idx, sp_idx = _bucket_indices(image)
    bi2 = bi_idx.reshape(IROWS, 128)
    sp2 = sp_idx.reshape(IROWS, 128)
    z = jnp.zeros((N, CP), jnp.float32)
    q = _tc_iter(z, upad, z, z)
    for _ in range(NUM_ITER):
        bg, sg, _, _ = _sc_splat_slice()(q, bi2, sp2)
        q = _tc_iter(q, upad, bg, sg)
    return q[:, :C].reshape(H, W, C)
